# baseline (speedup 1.0000x reference)
import functools

import jax
import jax.numpy as jnp
from jax import lax
from jax.experimental import pallas as pl
from jax.experimental.pallas import tpu as pltpu
from jax.experimental.pallas import tpu_sc as plsc

VOCAB_ = 1000000
SEQ_ = 200
BATCH_ = 4096
EMB_ = 32

NUM_WORKERS = 32
PACK = 128 // EMB_
TOTAL = BATCH_ * SEQ_
BLK = 128
UNITS = TOTAL // BLK
UNITS_PER_W = UNITS // NUM_WORKERS
BLOCKS_PER_SEQ = BATCH_ // BLK
KGROUPS = BLK // 16


def _sc_lookup(tok_t, resh):
  mesh = plsc.VectorSubcoreMesh(core_axis_name="c", subcore_axis_name="s")

  @functools.partial(
      pl.kernel,
      out_type=jax.ShapeDtypeStruct((SEQ_, EMB_, BATCH_), jnp.float32),
      mesh=mesh,
      scratch_types=[
          pltpu.VMEM((UNITS_PER_W * BLK,), jnp.int32),
          pltpu.VMEM((BLK,), jnp.int32),
          pltpu.VMEM((BLK,), jnp.int32),
          pltpu.VMEM((BLK,), jnp.int32),
          pltpu.VMEM((BLK,), jnp.int32),
          pltpu.VMEM((BLK, 128), jnp.float32),
          pltpu.VMEM((BLK, 128), jnp.float32),
          pltpu.VMEM((BLK, 128), jnp.float32),
          pltpu.VMEM((BLK, 128), jnp.float32),
          pltpu.VMEM((EMB_, BLK), jnp.float32),
          pltpu.VMEM((EMB_, BLK), jnp.float32),
          pltpu.SemaphoreType.DMA,
          pltpu.SemaphoreType.DMA,
          pltpu.SemaphoreType.DMA,
          pltpu.SemaphoreType.DMA,
          pltpu.SemaphoreType.DMA,
          pltpu.SemaphoreType.DMA,
      ],
      compiler_params=pltpu.CompilerParams(needs_layout_passes=False),
  )
  def body(tok_hbm, tab_hbm, out_hbm, idx_all, i40, i41, i42, i43, g0, g1,
           g2, g3, t0, t1, gs0, gs1, gs2, gs3, ws0, ws1):
    idx4 = (i40, i41, i42, i43)
    g = (g0, g1, g2, g3)
    tb_ = (t0, t1)
    gsem = (gs0, gs1, gs2, gs3)
    wsem = (ws0, ws1)
    wid = lax.axis_index("s") * 2 + lax.axis_index("c")
    ubase = wid * UNITS_PER_W

    pltpu.sync_copy(tok_hbm.at[pl.ds(ubase * BLK, UNITS_PER_W * BLK)],
                    idx_all)

    iota = lax.iota(jnp.int32, 16)
    rowvec = [iota + 16 * k for k in range(KGROUPS)]

    def compute_idx(u, b):
      off = u * BLK
      for k in range(KGROUPS):
        v = idx_all[pl.ds(off + 16 * k, 16)]
        idx4[b][pl.ds(16 * k, 16)] = lax.shift_right_logical(v, 2)

    def start_gather(u, b):
      compute_idx(u, b)
      pltpu.async_copy(tab_hbm.at[idx4[b]], g[b], gsem[b])

    def wait_gather(b):
      pltpu.make_async_copy(tab_hbm.at[idx4[b]], g[b], gsem[b]).wait()

    def transpose(u, b, tbuf):
      off = u * BLK
      for k in range(KGROUPS):
        v = idx_all[pl.ds(off + 16 * k, 16)]
        colbase = lax.shift_left(jnp.bitwise_and(v, PACK - 1), 5)
        vals = [plsc.load_gather(g[b], [rowvec[k], colbase + c])
                for c in range(EMB_)]
        for c in range(EMB_):
          tb_[tbuf][c, pl.ds(16 * k, 16)] = vals[c]

    def out_ref(gu):
      s = gu // BLOCKS_PER_SEQ
      tbk = gu % BLOCKS_PER_SEQ
      return out_hbm.at[s, :, pl.ds(tbk * BLK, BLK)]

    def wait_write(gu, b):
      pltpu.make_async_copy(tb_[b], out_ref(gu), wsem[b]).wait()

    for b in range(4):
      start_gather(b, b)

    def step(it, carry):
      for b in range(4):
        u = 4 * it + b
        gu = ubase + u
        tbuf = b % 2

        @pl.when(u >= 2)
        def _():
          wait_write(gu, tbuf)

        wait_gather(b)
        transpose(u, b, tbuf)
        j = u + 4

        @pl.when(j < UNITS_PER_W)
        def _():
          start_gather(j, b)

        pltpu.async_copy(tb_[tbuf], out_ref(gu), wsem[tbuf])
      return carry

    lax.fori_loop(0, UNITS_PER_W // 4, step, 0)
    for b in range(2):
      wait_write(ubase, b)

  return body(tok_t, resh)


def kernel(tokens, embedding):
  tok_t = jnp.transpose(tokens).reshape(TOTAL).astype(jnp.int32)
  resh = embedding.reshape(VOCAB_ // PACK, 128)
  r = _sc_lookup(tok_t, resh)
  return jnp.transpose(r, (2, 0, 1))


# speedup vs baseline: 1.0259x; 1.0259x over previous
import functools

import jax
import jax.numpy as jnp
from jax import lax
from jax.experimental import pallas as pl
from jax.experimental.pallas import tpu as pltpu
from jax.experimental.pallas import tpu_sc as plsc

VOCAB_ = 1000000
SEQ_ = 200
BATCH_ = 4096
EMB_ = 32

NUM_WORKERS = 32
PACK = 128 // EMB_
TOTAL = BATCH_ * SEQ_
BLK = 256
UNITS = TOTAL // BLK
UNITS_PER_W = UNITS // NUM_WORKERS
BLOCKS_PER_SEQ = BATCH_ // BLK
KGROUPS = BLK // 16


def _sc_lookup(tok_t, resh):
  mesh = plsc.VectorSubcoreMesh(core_axis_name="c", subcore_axis_name="s")

  @functools.partial(
      pl.kernel,
      out_type=jax.ShapeDtypeStruct((SEQ_, EMB_, BATCH_), jnp.float32),
      mesh=mesh,
      scratch_types=[
          pltpu.VMEM((UNITS_PER_W * BLK,), jnp.int32),
          pltpu.VMEM((BLK,), jnp.int32),
          pltpu.VMEM((BLK,), jnp.int32),
          pltpu.VMEM((BLK, 128), jnp.float32),
          pltpu.VMEM((BLK, 128), jnp.float32),
          pltpu.VMEM((EMB_, BLK), jnp.float32),
          pltpu.VMEM((EMB_, BLK), jnp.float32),
          pltpu.SemaphoreType.DMA,
          pltpu.SemaphoreType.DMA,
          pltpu.SemaphoreType.DMA,
          pltpu.SemaphoreType.DMA,
      ],
      compiler_params=pltpu.CompilerParams(needs_layout_passes=False),
  )
  def body(tok_hbm, tab_hbm, out_hbm, idx_all, i40, i41, g0, g1, t0, t1,
           gs0, gs1, ws0, ws1):
    idx4 = (i40, i41)
    g = (g0, g1)
    tb_ = (t0, t1)
    gsem = (gs0, gs1)
    wsem = (ws0, ws1)
    wid = lax.axis_index("s") * 2 + lax.axis_index("c")
    ubase = wid * UNITS_PER_W

    pltpu.sync_copy(tok_hbm.at[pl.ds(ubase * BLK, UNITS_PER_W * BLK)],
                    idx_all)

    iota = lax.iota(jnp.int32, 16)
    rowvec = [iota + 16 * k for k in range(KGROUPS)]

    def compute_idx(u, b):
      off = u * BLK
      for k in range(KGROUPS):
        v = idx_all[pl.ds(off + 16 * k, 16)]
        idx4[b][pl.ds(16 * k, 16)] = lax.shift_right_logical(v, 2)

    def start_gather(u, b):
      compute_idx(u, b)
      pltpu.async_copy(tab_hbm.at[idx4[b]], g[b], gsem[b])

    def wait_gather(b):
      pltpu.make_async_copy(tab_hbm.at[idx4[b]], g[b], gsem[b]).wait()

    def transpose(u, b):
      off = u * BLK
      for k in range(KGROUPS):
        v = idx_all[pl.ds(off + 16 * k, 16)]
        colbase = lax.shift_left(jnp.bitwise_and(v, PACK - 1), 5)
        vals = [plsc.load_gather(g[b], [rowvec[k], colbase + c])
                for c in range(EMB_)]
        for c in range(EMB_):
          tb_[b][c, pl.ds(16 * k, 16)] = vals[c]

    def out_ref(gu):
      s = gu // BLOCKS_PER_SEQ
      tbk = gu % BLOCKS_PER_SEQ
      return out_hbm.at[s, :, pl.ds(tbk * BLK, BLK)]

    def wait_write(gu, b):
      pltpu.make_async_copy(tb_[b], out_ref(gu), wsem[b]).wait()

    start_gather(0, 0)
    start_gather(1, 1)

    def step(it, carry):
      for b in range(2):
        u = 2 * it + b
        gu = ubase + u

        @pl.when(u >= 2)
        def _():
          wait_write(gu, b)

        wait_gather(b)
        transpose(u, b)
        j = u + 2

        @pl.when(j < UNITS_PER_W)
        def _():
          start_gather(j, b)

        pltpu.async_copy(tb_[b], out_ref(gu), wsem[b])
      return carry

    lax.fori_loop(0, UNITS_PER_W // 2, step, 0)
    for b in range(2):
      wait_write(ubase, b)

  return body(tok_t, resh)


def kernel(tokens, embedding):
  tok_t = jnp.transpose(tokens).reshape(TOTAL).astype(jnp.int32)
  resh = embedding.reshape(VOCAB_ // PACK, 128)
  r = _sc_lookup(tok_t, resh)
  return jnp.transpose(r, (2, 0, 1))


# speedup vs baseline: 1.0392x; 1.0129x over previous
import functools

import jax
import jax.numpy as jnp
from jax import lax
from jax.experimental import pallas as pl
from jax.experimental.pallas import tpu as pltpu
from jax.experimental.pallas import tpu_sc as plsc

VOCAB_ = 1000000
SEQ_ = 200
BATCH_ = 4096
EMB_ = 32

NUM_WORKERS = 32
PACK = 128 // EMB_
TOTAL = BATCH_ * SEQ_
BLK = 128
UNITS = TOTAL // BLK
UNITS_PER_W = UNITS // NUM_WORKERS
BLOCKS_PER_SEQ = BATCH_ // BLK
KGROUPS = BLK // 16


def _sc_lookup(tok_t, resh):
  mesh = plsc.VectorSubcoreMesh(core_axis_name="c", subcore_axis_name="s")

  @functools.partial(
      pl.kernel,
      out_type=jax.ShapeDtypeStruct((SEQ_, EMB_, BATCH_), jnp.float32),
      mesh=mesh,
      scratch_types=[
          pltpu.VMEM((UNITS_PER_W * BLK,), jnp.int32),
          pltpu.VMEM((BLK,), jnp.int32),
          pltpu.VMEM((BLK,), jnp.int32),
          pltpu.VMEM((BLK, 128), jnp.float32),
          pltpu.VMEM((BLK, 128), jnp.float32),
          pltpu.VMEM((EMB_, BLK), jnp.float32),
          pltpu.VMEM((EMB_, BLK), jnp.float32),
          pltpu.SemaphoreType.DMA,
          pltpu.SemaphoreType.DMA,
          pltpu.SemaphoreType.DMA,
          pltpu.SemaphoreType.DMA,
      ],
      compiler_params=pltpu.CompilerParams(needs_layout_passes=False),
  )
  def body(tok_hbm, tab_hbm, out_hbm, idx_all, i40, i41, g0, g1, t0, t1,
           gs0, gs1, ws0, ws1):
    idx4 = (i40, i41)
    g = (g0, g1)
    tb_ = (t0, t1)
    gsem = (gs0, gs1)
    wsem = (ws0, ws1)
    wid = lax.axis_index("s") * 2 + lax.axis_index("c")
    ubase = wid * UNITS_PER_W

    pltpu.sync_copy(tok_hbm.at[pl.ds(ubase * BLK, UNITS_PER_W * BLK)],
                    idx_all)

    iota = lax.iota(jnp.int32, 16)
    rowvec = [iota + 16 * k for k in range(KGROUPS)]

    def compute_idx(u, b):
      off = u * BLK
      for k in range(KGROUPS):
        v = idx_all[pl.ds(off + 16 * k, 16)]
        idx4[b][pl.ds(16 * k, 16)] = lax.shift_right_logical(v, 2)

    def start_gather(u, b):
      compute_idx(u, b)
      pltpu.async_copy(tab_hbm.at[idx4[b]], g[b], gsem[b])

    def wait_gather(b):
      pltpu.make_async_copy(tab_hbm.at[idx4[b]], g[b], gsem[b]).wait()

    def transpose(u, b):
      off = u * BLK
      for k in range(KGROUPS):
        v = idx_all[pl.ds(off + 16 * k, 16)]
        colbase = lax.shift_left(jnp.bitwise_and(v, PACK - 1), 5)
        vals = [plsc.load_gather(g[b], [rowvec[k], colbase + c])
                for c in range(EMB_)]
        for c in range(EMB_):
          tb_[b][c, pl.ds(16 * k, 16)] = vals[c]

    def out_ref(gu):
      s = gu // BLOCKS_PER_SEQ
      tbk = gu % BLOCKS_PER_SEQ
      return out_hbm.at[s, :, pl.ds(tbk * BLK, BLK)]

    def wait_write(gu, b):
      pltpu.make_async_copy(tb_[b], out_ref(gu), wsem[b]).wait()

    start_gather(0, 0)
    start_gather(1, 1)

    def step(it, carry):
      for b in range(2):
        u = 2 * it + b
        gu = ubase + u

        @pl.when(u >= 2)
        def _():
          wait_write(gu, b)

        wait_gather(b)
        transpose(u, b)
        j = u + 2

        @pl.when(j < UNITS_PER_W)
        def _():
          start_gather(j, b)

        pltpu.async_copy(tb_[b], out_ref(gu), wsem[b])
      return carry

    lax.fori_loop(0, UNITS_PER_W // 2, step, 0)
    for b in range(2):
      wait_write(ubase, b)

  return body(tok_t, resh)


def kernel(tokens, embedding):
  tok_t = jnp.transpose(tokens).reshape(TOTAL).astype(jnp.int32)
  resh = embedding.reshape(VOCAB_ // PACK, 128)
  r = _sc_lookup(tok_t, resh)
  return jnp.transpose(r, (2, 0, 1))


# speedup vs baseline: 1.0456x; 1.0061x over previous
import functools

import jax
import jax.numpy as jnp
from jax import lax
from jax.experimental import pallas as pl
from jax.experimental.pallas import tpu as pltpu
from jax.experimental.pallas import tpu_sc as plsc

VOCAB_ = 1000000
SEQ_ = 200
BATCH_ = 4096
EMB_ = 32

NUM_WORKERS = 32
PACK = 128 // EMB_
TOTAL = BATCH_ * SEQ_
BLK = 128
UNITS = TOTAL // BLK
UNITS_PER_W = UNITS // NUM_WORKERS
BLOCKS_PER_SEQ = BATCH_ // BLK
KGROUPS = BLK // 16


def _sc_lookup(tok_t, resh):
  mesh = plsc.VectorSubcoreMesh(core_axis_name="c", subcore_axis_name="s")

  @functools.partial(
      pl.kernel,
      out_type=jax.ShapeDtypeStruct((SEQ_, EMB_, BATCH_), jnp.float32),
      mesh=mesh,
      scratch_types=[
          pltpu.VMEM((UNITS_PER_W * BLK,), jnp.int32),
          pltpu.VMEM((BLK,), jnp.int32),
          pltpu.VMEM((BLK,), jnp.int32),
          pltpu.VMEM((BLK,), jnp.int32),
          pltpu.VMEM((BLK,), jnp.int32),
          pltpu.VMEM((BLK, 128), jnp.float32),
          pltpu.VMEM((BLK, 128), jnp.float32),
          pltpu.VMEM((EMB_, BLK), jnp.float32),
          pltpu.VMEM((EMB_, BLK), jnp.float32),
          pltpu.SemaphoreType.DMA,
          pltpu.SemaphoreType.DMA,
          pltpu.SemaphoreType.DMA,
          pltpu.SemaphoreType.DMA,
      ],
      compiler_params=pltpu.CompilerParams(needs_layout_passes=False),
  )
  def body(tok_hbm, tab_hbm, out_hbm, idx_all, i40, i41, cb0, cb1, g0, g1,
           t0, t1, gs0, gs1, ws0, ws1):
    idx4 = (i40, i41)
    cbs = (cb0, cb1)
    g = (g0, g1)
    tb_ = (t0, t1)
    gsem = (gs0, gs1)
    wsem = (ws0, ws1)
    wid = lax.axis_index("s") * 2 + lax.axis_index("c")
    ubase = wid * UNITS_PER_W

    pltpu.sync_copy(tok_hbm.at[pl.ds(ubase * BLK, UNITS_PER_W * BLK)],
                    idx_all)

    iota = lax.iota(jnp.int32, 16)
    rowvec = [iota + 16 * k for k in range(KGROUPS)]

    def compute_idx(u, b):
      off = u * BLK
      for k in range(KGROUPS):
        v = idx_all[pl.ds(off + 16 * k, 16)]
        idx4[b][pl.ds(16 * k, 16)] = lax.shift_right_logical(v, 2)
        cbs[b][pl.ds(16 * k, 16)] = lax.shift_left(
            jnp.bitwise_and(v, PACK - 1), 5)

    def start_gather(u, b):
      compute_idx(u, b)
      pltpu.async_copy(tab_hbm.at[idx4[b]], g[b], gsem[b])

    def wait_gather(b):
      pltpu.make_async_copy(tab_hbm.at[idx4[b]], g[b], gsem[b]).wait()

    def transpose(u, b):
      for k in range(KGROUPS):
        colbase = cbs[b][pl.ds(16 * k, 16)]
        vals = [plsc.load_gather(g[b], [rowvec[k], colbase + c])
                for c in range(EMB_)]
        for c in range(EMB_):
          tb_[b][c, pl.ds(16 * k, 16)] = vals[c]

    def out_ref(gu):
      s = gu // BLOCKS_PER_SEQ
      tbk = gu % BLOCKS_PER_SEQ
      return out_hbm.at[s, :, pl.ds(tbk * BLK, BLK)]

    def wait_write(gu, b):
      pltpu.make_async_copy(tb_[b], out_ref(gu), wsem[b]).wait()

    start_gather(0, 0)
    start_gather(1, 1)

    def step(it, carry):
      for b in range(2):
        u = 2 * it + b
        gu = ubase + u

        @pl.when(u >= 2)
        def _():
          wait_write(gu, b)

        wait_gather(b)
        transpose(u, b)
        j = u + 2

        @pl.when(j < UNITS_PER_W)
        def _():
          start_gather(j, b)

        pltpu.async_copy(tb_[b], out_ref(gu), wsem[b])
      return carry

    lax.fori_loop(0, UNITS_PER_W // 2, step, 0)
    for b in range(2):
      wait_write(ubase, b)

  return body(tok_t, resh)


def kernel(tokens, embedding):
  tok_t = jnp.transpose(tokens).reshape(TOTAL).astype(jnp.int32)
  resh = embedding.reshape(VOCAB_ // PACK, 128)
  r = _sc_lookup(tok_t, resh)
  return jnp.transpose(r, (2, 0, 1))
